# Initial kernel scaffold; baseline (speedup 1.0000x reference)
import jax
import jax.numpy as jnp
from jax.experimental import pallas as pl


def kernel(x, f, W1, b1, W2, gamma, beta):
    raise NotImplementedError("write your pallas kernel here")


# speedup vs baseline: 15.4391x; 15.4391x over previous
import functools

import jax
import jax.numpy as jnp
from jax import lax
from jax.experimental import pallas as pl
from jax.experimental.pallas import tpu as pltpu
from jax.experimental.pallas import tpu_sc as plsc

B = 8
N = 2048
K = 16
C1 = 64
C2 = 256
NGROUP = 4
NPOINT = 512
ROWS = 256


def _k1_body(f_ref, w1_ref, b1_ref, w2_ref, u_ref, v_ref):
    fb = f_ref[0]
    f64 = lax.dot_general(fb, w1_ref[...], (((0,), (0,)), ((), ())),
                          preferred_element_type=jnp.float32)
    f64 = f64 + b1_ref[...]
    w2a = w2_ref[0:C1, :]
    w2b = w2_ref[C1:2 * C1, :]
    u_ref[0] = lax.dot_general(f64, w2a, (((1,), (0,)), ((), ())),
                               preferred_element_type=jnp.float32)
    v_ref[0] = lax.dot_general(f64, w2b - w2a, (((1,), (0,)), ((), ())),
                               preferred_element_type=jnp.float32)


def _run_k1(f, W1, b1, W2):
    return pl.pallas_call(
        _k1_body,
        grid=(B,),
        in_specs=[
            pl.BlockSpec((1, 3, N), lambda b: (b, 0, 0)),
            pl.BlockSpec((3, C1), lambda b: (0, 0)),
            pl.BlockSpec((1, C1), lambda b: (0, 0)),
            pl.BlockSpec((2 * C1, C2), lambda b: (0, 0)),
        ],
        out_specs=[
            pl.BlockSpec((1, N, C2), lambda b: (b, 0, 0)),
            pl.BlockSpec((1, N, C2), lambda b: (b, 0, 0)),
        ],
        out_shape=[
            jax.ShapeDtypeStruct((B, N, C2), jnp.float32),
            jax.ShapeDtypeStruct((B, N, C2), jnp.float32),
        ],
    )(f, W1, b1.reshape(1, C1), W2)


def _k2_body(xr_ref, x_ref, idx_ref):
    xr = xr_ref[0]
    xb = x_ref[0]
    g = lax.dot_general(xr, xb, (((1,), (0,)), ((), ())),
                        preferred_element_type=jnp.float32)
    xi2 = jnp.sum(xr * xr, axis=1, keepdims=True)
    xj2 = jnp.sum(xb * xb, axis=0, keepdims=True)
    cur = (-2.0 * g + xi2) + xj2
    iota = lax.broadcasted_iota(jnp.int32, (ROWS, N), 1)
    cols = []
    for _ in range(K):
        m = jnp.min(cur, axis=1, keepdims=True)
        cand = jnp.where(cur == m, iota, N)
        sel = jnp.min(cand, axis=1, keepdims=True)
        cols.append(sel)
        cur = jnp.where(iota == sel, jnp.inf, cur)
    idx_ref[0] = jnp.concatenate(cols, axis=1)


def _run_k2(x, xt):
    return pl.pallas_call(
        _k2_body,
        grid=(B, N // ROWS),
        in_specs=[
            pl.BlockSpec((1, ROWS, 3), lambda b, r: (b, r, 0)),
            pl.BlockSpec((1, 3, N), lambda b, r: (b, 0, 0)),
        ],
        out_specs=pl.BlockSpec((1, ROWS, K), lambda b, r: (b, r, 0)),
        out_shape=jax.ShapeDtypeStruct((B, N, K), jnp.int32),
    )(xt, x)


def _k3_body(x_ref, out_ref):
    x0 = x_ref[:, 0, :]
    x1 = x_ref[:, 1, :]
    x2 = x_ref[:, 2, :]
    iota_i = lax.broadcasted_iota(jnp.int32, (B, N), 1)
    iota_np = lax.broadcasted_iota(jnp.int32, (B, NPOINT), 1)

    def body(i, carry):
        dist, far, fars = carry
        far_mat = far + 0 * iota_np
        fars = jnp.where(iota_np == i, far_mat, fars)
        oh = iota_i == far
        cx = jnp.sum(jnp.where(oh, x0, 0.0), axis=1, keepdims=True)
        cy = jnp.sum(jnp.where(oh, x1, 0.0), axis=1, keepdims=True)
        cz = jnp.sum(jnp.where(oh, x2, 0.0), axis=1, keepdims=True)
        dx = x0 - cx
        dy = x1 - cy
        dz = x2 - cz
        d = (dx * dx + dy * dy) + dz * dz
        dist = jnp.minimum(dist, d)
        m = jnp.max(dist, axis=1, keepdims=True)
        cand = jnp.where(dist == m, iota_i, N)
        far_new = jnp.min(cand, axis=1, keepdims=True)
        return dist, far_new, fars

    init = (1e10 + 0.0 * x0,
            (0.0 * jnp.min(x0, axis=1, keepdims=True)).astype(jnp.int32),
            (0.0 * x0[:, :NPOINT]).astype(jnp.int32))
    _, _, fars = lax.fori_loop(0, NPOINT, body, init)
    out_ref[...] = fars


def _run_k3(x):
    return pl.pallas_call(
        _k3_body,
        grid=(1,),
        in_specs=[pl.BlockSpec((B, 3, N), lambda i: (0, 0, 0))],
        out_specs=pl.BlockSpec((B, NPOINT), lambda i: (0, 0)),
        out_shape=jax.ShapeDtypeStruct((B, NPOINT), jnp.int32),
    )(x)


PB = 8
EB = PB * K


def _k4_sc_body(u_hbm, v_hbm, idx_hbm, mu_hbm, cross_hbm, sumu_hbm,
                sumusq_hbm, idx_v, idxg_v, rows_v, v_buf, mu_buf, cross_v,
                sumu_v, sumusq_v, sem):
    info = plsc.get_sparse_core_info()
    nc = info.num_cores
    w = lax.axis_index("s") * nc + lax.axis_index("c")
    ppw = (B * N) // (nc * info.num_subcores)
    base = (w // (N // ppw)) * N

    zero16f = jnp.zeros((16,), jnp.float32)
    for j in range(C2 // 16):
        cross_v[pl.ds(j * 16, 16)] = zero16f
        sumu_v[pl.ds(j * 16, 16)] = zero16f
        sumusq_v[pl.ds(j * 16, 16)] = zero16f

    def block_body(blk, _):
        p0 = w * ppw + blk * PB
        pltpu.sync_copy(idx_hbm.at[pl.ds(p0 * K, EB)], idx_v)
        for j in range(EB // 16):
            t = idx_v[pl.ds(j * 16, 16)]
            idxg_v[pl.ds(j * 16, 16)] = t + base
        cp_v = pltpu.async_copy(v_hbm.at[pl.ds(p0, PB)], v_buf, sem)
        pltpu.async_copy(u_hbm.at[idxg_v], rows_v, sem).wait()
        cp_v.wait()

        def chunk_body(c, _):
            co = c * 16
            cacc = cross_v[pl.ds(co, 16)]
            uacc = sumu_v[pl.ds(co, 16)]
            qacc = sumusq_v[pl.ds(co, 16)]
            for p in range(PB):
                r0 = p * K
                m = rows_v[r0, pl.ds(co, 16)]
                s = m
                q = m * m
                for r in range(1, K):
                    t = rows_v[r0 + r, pl.ds(co, 16)]
                    m = jnp.maximum(m, t)
                    s = s + t
                    q = q + t * t
                mu_buf[p, pl.ds(co, 16)] = m
                cacc = cacc + s * v_buf[p, pl.ds(co, 16)]
                uacc = uacc + s
                qacc = qacc + q
            cross_v[pl.ds(co, 16)] = cacc
            sumu_v[pl.ds(co, 16)] = uacc
            sumusq_v[pl.ds(co, 16)] = qacc
            return 0

        lax.fori_loop(0, C2 // 16, chunk_body, 0)
        pltpu.sync_copy(mu_buf, mu_hbm.at[pl.ds(p0, PB)])
        return 0

    lax.fori_loop(0, ppw // PB, block_body, 0)
    pltpu.sync_copy(cross_v, cross_hbm.at[w])
    pltpu.sync_copy(sumu_v, sumu_hbm.at[w])
    pltpu.sync_copy(sumusq_v, sumusq_hbm.at[w])


def _run_k4(u_flat, v_flat, idx_flat):
    mesh = plsc.VectorSubcoreMesh(core_axis_name="c", subcore_axis_name="s")
    kfn = functools.partial(
        pl.kernel,
        mesh=mesh,
        out_type=[
            jax.ShapeDtypeStruct((B * N, C2), jnp.float32),
            jax.ShapeDtypeStruct((32, C2), jnp.float32),
            jax.ShapeDtypeStruct((32, C2), jnp.float32),
            jax.ShapeDtypeStruct((32, C2), jnp.float32),
        ],
        scratch_types=[
            pltpu.VMEM((EB,), jnp.int32),
            pltpu.VMEM((EB,), jnp.int32),
            pltpu.VMEM((EB, C2), jnp.float32),
            pltpu.VMEM((PB, C2), jnp.float32),
            pltpu.VMEM((PB, C2), jnp.float32),
            pltpu.VMEM((C2,), jnp.float32),
            pltpu.VMEM((C2,), jnp.float32),
            pltpu.VMEM((C2,), jnp.float32),
            pltpu.SemaphoreType.DMA,
        ],
    )(_k4_sc_body)
    return kfn(u_flat, v_flat, idx_flat)


def _k5_body(u_ref, v_ref, mu_ref, sumu_ref, sumusq_ref, cross_ref, fps_ref,
             x_ref, g_ref, bt_ref, coor_ref, newx_ref):
    v = v_ref[0]
    mu = mu_ref[0]
    cross = jnp.sum(cross_ref[0], axis=0, keepdims=True)
    hi = jax.lax.Precision.HIGHEST
    sum_u = jnp.sum(sumu_ref[0], axis=0, keepdims=True)
    sum_usq = jnp.sum(sumusq_ref[0], axis=0, keepdims=True)
    sum_v = jnp.sum(v, axis=0, keepdims=True)
    sum_vsq = jnp.sum(v * v, axis=0, keepdims=True)
    s1 = sum_u + float(K) * sum_v
    s2 = sum_usq + 2.0 * cross + float(K) * sum_vsq
    gsel = (lax.broadcasted_iota(jnp.int32, (C2, NGROUP), 0) // (C2 // NGROUP)
            == lax.broadcasted_iota(jnp.int32, (C2, NGROUP), 1)
            ).astype(jnp.float32)
    s1g = lax.dot_general(s1, gsel, (((1,), (0,)), ((), ())),
                          precision=hi, preferred_element_type=jnp.float32)
    s2g = lax.dot_general(s2, gsel, (((1,), (0,)), ((), ())),
                          precision=hi, preferred_element_type=jnp.float32)
    cnt_total = float((C2 // NGROUP) * N * K)
    mean_g = s1g / cnt_total
    var_g = s2g / cnt_total - mean_g * mean_g
    rstd_g = lax.rsqrt(var_g + 1e-5)
    gselt = (lax.broadcasted_iota(jnp.int32, (NGROUP, C2), 0) ==
             lax.broadcasted_iota(jnp.int32, (NGROUP, C2), 1) // (C2 // NGROUP)
             ).astype(jnp.float32)
    rstd = lax.dot_general(rstd_g, gselt, (((1,), (0,)), ((), ())),
                           precision=hi, preferred_element_type=jnp.float32)
    mean = lax.dot_general(mean_g, gselt, (((1,), (0,)), ((), ())),
                           precision=hi, preferred_element_type=jnp.float32)
    scale = g_ref[...] * rstd
    shift = bt_ref[...] - mean * scale
    act = (mu + v) * scale + shift
    act = jnp.where(act >= 0.0, act, 0.2 * act)
    frow = fps_ref[0]
    pt = (lax.broadcasted_iota(jnp.int32, (N, NPOINT), 0) == frow
          ).astype(jnp.float32)
    newx_ref[0] = lax.dot_general(act, pt, (((0,), (0,)), ((), ())),
                                  precision=hi,
                                  preferred_element_type=jnp.float32)
    xb = x_ref[0]
    coor_ref[0] = lax.dot_general(xb, pt, (((1,), (0,)), ((), ())),
                                  precision=hi,
                                  preferred_element_type=jnp.float32)


def _run_k5(u, v, mu, sumu_part, sumusq_part, cross_part, fps_sm, x, gamma,
            beta):
    return pl.pallas_call(
        _k5_body,
        grid=(B,),
        in_specs=[
            pl.BlockSpec((1, N, C2), lambda b: (b, 0, 0)),
            pl.BlockSpec((1, N, C2), lambda b: (b, 0, 0)),
            pl.BlockSpec((1, N, C2), lambda b: (b, 0, 0)),
            pl.BlockSpec((1, 4, C2), lambda b: (b, 0, 0)),
            pl.BlockSpec((1, 4, C2), lambda b: (b, 0, 0)),
            pl.BlockSpec((1, 4, C2), lambda b: (b, 0, 0)),
            pl.BlockSpec((1, 1, NPOINT), lambda b: (b, 0, 0)),
            pl.BlockSpec((1, 3, N), lambda b: (b, 0, 0)),
            pl.BlockSpec((1, C2), lambda b: (0, 0)),
            pl.BlockSpec((1, C2), lambda b: (0, 0)),
        ],
        out_specs=[
            pl.BlockSpec((1, 3, NPOINT), lambda b: (b, 0, 0)),
            pl.BlockSpec((1, C2, NPOINT), lambda b: (b, 0, 0)),
        ],
        out_shape=[
            jax.ShapeDtypeStruct((B, 3, NPOINT), jnp.float32),
            jax.ShapeDtypeStruct((B, C2, NPOINT), jnp.float32),
        ],
    )(u, v, mu, sumu_part.reshape(B, 4, C2), sumusq_part.reshape(B, 4, C2),
      cross_part.reshape(B, 4, C2), fps_sm.reshape(B, 1, NPOINT), x,
      gamma.reshape(1, C2), beta.reshape(1, C2))


def _k4_jnp_debug(u_flat, v_flat, idx_flat):
    u = u_flat.reshape(B, N, C2)
    v = v_flat.reshape(B, N, C2)
    idx = idx_flat.reshape(B, N, K)
    gath = jax.vmap(lambda uu, ii: uu[ii])(u, idx)
    mu = jnp.max(gath, axis=2)
    su = jnp.sum(gath, axis=2)
    cross_b = jnp.einsum("bnd,bnd->bd", su, v)
    sumu_b = jnp.sum(su, axis=1)
    sumusq_b = jnp.einsum("bnkd->bd", gath * gath)
    cross_part = jnp.zeros((32, C2), jnp.float32).at[0::4].set(cross_b)
    sumu_part = jnp.zeros((32, C2), jnp.float32).at[0::4].set(sumu_b)
    sumusq_part = jnp.zeros((32, C2), jnp.float32).at[0::4].set(sumusq_b)
    return mu.reshape(B * N, C2), cross_part, sumu_part, sumusq_part


def kernel(x, f, W1, b1, W2, gamma, beta):
    xt = jnp.swapaxes(x, 1, 2)
    u, v = _run_k1(f, W1, b1, W2)
    idx = _run_k2(x, xt)
    fps_sm = _run_k3(x)
    mu_flat, cross_part, sumu_part, sumusq_part = _run_k4(
        u.reshape(B * N, C2), v.reshape(B * N, C2),
        idx.reshape(B * N * K))
    mu = mu_flat.reshape(B, N, C2)
    new_coor, new_x = _run_k5(u, v, mu, sumu_part, sumusq_part, cross_part,
                              fps_sm, x, gamma, beta)
    return (new_coor, new_x, fps_sm)


# speedup vs baseline: 20.2850x; 1.3139x over previous
import functools

import jax
import jax.numpy as jnp
from jax import lax
from jax.experimental import pallas as pl
from jax.experimental.pallas import tpu as pltpu
from jax.experimental.pallas import tpu_sc as plsc

B = 8
N = 2048
K = 16
C1 = 64
C2 = 256
NGROUP = 4
NPOINT = 512
ROWS = 256


def _k1_body(f_ref, w1_ref, b1_ref, w2_ref, u_ref, v_ref):
    fb = f_ref[0]
    f64 = lax.dot_general(fb, w1_ref[...], (((0,), (0,)), ((), ())),
                          preferred_element_type=jnp.float32)
    f64 = f64 + b1_ref[...]
    w2a = w2_ref[0:C1, :]
    w2b = w2_ref[C1:2 * C1, :]
    u_ref[0] = lax.dot_general(f64, w2a, (((1,), (0,)), ((), ())),
                               preferred_element_type=jnp.float32)
    v_ref[0] = lax.dot_general(f64, w2b - w2a, (((1,), (0,)), ((), ())),
                               preferred_element_type=jnp.float32)


def _run_k1(f, W1, b1, W2):
    return pl.pallas_call(
        _k1_body,
        grid=(B,),
        in_specs=[
            pl.BlockSpec((1, 3, N), lambda b: (b, 0, 0)),
            pl.BlockSpec((3, C1), lambda b: (0, 0)),
            pl.BlockSpec((1, C1), lambda b: (0, 0)),
            pl.BlockSpec((2 * C1, C2), lambda b: (0, 0)),
        ],
        out_specs=[
            pl.BlockSpec((1, N, C2), lambda b: (b, 0, 0)),
            pl.BlockSpec((1, N, C2), lambda b: (b, 0, 0)),
        ],
        out_shape=[
            jax.ShapeDtypeStruct((B, N, C2), jnp.float32),
            jax.ShapeDtypeStruct((B, N, C2), jnp.float32),
        ],
    )(f, W1, b1.reshape(1, C1), W2)


def _k2_body(xr_ref, x_ref, idx_ref):
    xr = xr_ref[0]
    xb = x_ref[0]
    g = lax.dot_general(xr, xb, (((1,), (0,)), ((), ())),
                        preferred_element_type=jnp.float32)
    xi2 = jnp.sum(xr * xr, axis=1, keepdims=True)
    xj2 = jnp.sum(xb * xb, axis=0, keepdims=True)
    cur = (-2.0 * g + xi2) + xj2
    iota_f = lax.broadcasted_iota(jnp.int32, (ROWS, N), 1).astype(jnp.float32)
    cols = []
    for _ in range(K):
        m = jnp.min(cur, axis=1, keepdims=True)
        cand = jnp.where(cur == m, iota_f, float(N))
        sel = jnp.min(cand, axis=1, keepdims=True)
        cols.append(sel.astype(jnp.int32))
        cur = jnp.where(iota_f == sel, jnp.inf, cur)
    idx_ref[0] = jnp.concatenate(cols, axis=1)


def _run_k2(x, xt):
    return pl.pallas_call(
        _k2_body,
        grid=(B, N // ROWS),
        in_specs=[
            pl.BlockSpec((1, ROWS, 3), lambda b, r: (b, r, 0)),
            pl.BlockSpec((1, 3, N), lambda b, r: (b, 0, 0)),
        ],
        out_specs=pl.BlockSpec((1, ROWS, K), lambda b, r: (b, r, 0)),
        out_shape=jax.ShapeDtypeStruct((B, N, K), jnp.int32),
    )(xt, x)


def _k3_body(x_ref, out_ref):
    x0 = x_ref[:, 0, :]
    x1 = x_ref[:, 1, :]
    x2 = x_ref[:, 2, :]
    iota_i = lax.broadcasted_iota(jnp.int32, (B, N), 1)
    iota_np = lax.broadcasted_iota(jnp.int32, (B, NPOINT), 1)

    def body(i, carry):
        dist, far, fars = carry
        far_mat = far + 0 * iota_np
        fars = jnp.where(iota_np == i, far_mat, fars)
        oh = iota_i == far
        cx = jnp.sum(jnp.where(oh, x0, 0.0), axis=1, keepdims=True)
        cy = jnp.sum(jnp.where(oh, x1, 0.0), axis=1, keepdims=True)
        cz = jnp.sum(jnp.where(oh, x2, 0.0), axis=1, keepdims=True)
        dx = x0 - cx
        dy = x1 - cy
        dz = x2 - cz
        d = (dx * dx + dy * dy) + dz * dz
        dist = jnp.minimum(dist, d)
        m = jnp.max(dist, axis=1, keepdims=True)
        cand = jnp.where(dist == m, iota_i, N)
        far_new = jnp.min(cand, axis=1, keepdims=True)
        return dist, far_new, fars

    init = (1e10 + 0.0 * x0,
            (0.0 * jnp.min(x0, axis=1, keepdims=True)).astype(jnp.int32),
            (0.0 * x0[:, :NPOINT]).astype(jnp.int32))
    _, _, fars = lax.fori_loop(0, NPOINT, body, init)
    out_ref[...] = fars


def _run_k3(x):
    return pl.pallas_call(
        _k3_body,
        grid=(1,),
        in_specs=[pl.BlockSpec((B, 3, N), lambda i: (0, 0, 0))],
        out_specs=pl.BlockSpec((B, NPOINT), lambda i: (0, 0)),
        out_shape=jax.ShapeDtypeStruct((B, NPOINT), jnp.int32),
    )(x)


PB = 8
EB = PB * K


def _k4_sc_body(u_hbm, v_hbm, idx_hbm, mu_hbm, cross_hbm, sumu_hbm,
                sumusq_hbm, idx0, idx1, idxg0, idxg1, rows0, rows1, vb0, vb1,
                mu_buf, cross_v, sumu_v, sumusq_v, si0, si1, sd0, sd1):
    info = plsc.get_sparse_core_info()
    nc = info.num_cores
    w = lax.axis_index("s") * nc + lax.axis_index("c")
    ppw = (B * N) // (nc * info.num_subcores)
    base = (w // (N // ppw)) * N
    nblk = ppw // PB
    bufs = [(idx0, idxg0, rows0, vb0, si0, sd0),
            (idx1, idxg1, rows1, vb1, si1, sd1)]

    zero16f = jnp.zeros((16,), jnp.float32)
    for j in range(C2 // 16):
        cross_v[pl.ds(j * 16, 16)] = zero16f
        sumu_v[pl.ds(j * 16, 16)] = zero16f
        sumusq_v[pl.ds(j * 16, 16)] = zero16f

    def fire_idx(g, k):
        idx_v, _, _, _, si, _ = bufs[k]
        pltpu.async_copy(idx_hbm.at[pl.ds((w * ppw + g * PB) * K, EB)],
                         idx_v, si)

    def wait_idx(g, k):
        idx_v, _, _, _, si, _ = bufs[k]
        pltpu.make_async_copy(idx_hbm.at[pl.ds((w * ppw + g * PB) * K, EB)],
                              idx_v, si).wait()

    def fire_dat(g, k):
        idx_v, idxg_v, rows_v, v_buf, _, sd = bufs[k]
        for j in range(EB // 16):
            t = idx_v[pl.ds(j * 16, 16)]
            idxg_v[pl.ds(j * 16, 16)] = t + base
        pltpu.async_copy(u_hbm.at[idxg_v], rows_v, sd)
        pltpu.async_copy(v_hbm.at[pl.ds(w * ppw + g * PB, PB)], v_buf, sd)

    def wait_dat(g, k):
        _, idxg_v, rows_v, v_buf, _, sd = bufs[k]
        pltpu.make_async_copy(u_hbm.at[idxg_v], rows_v, sd).wait()
        pltpu.make_async_copy(v_hbm.at[pl.ds(w * ppw + g * PB, PB)], v_buf,
                              sd).wait()

    def compute(g, k):
        _, _, rows_v, v_buf, _, _ = bufs[k]

        def chunk_body(c, _):
            co = c * 16
            cacc = cross_v[pl.ds(co, 16)]
            uacc = sumu_v[pl.ds(co, 16)]
            qacc = sumusq_v[pl.ds(co, 16)]
            for p in range(PB):
                r0 = p * K
                m = rows_v[r0, pl.ds(co, 16)]
                s = m
                q = m * m
                for r in range(1, K):
                    t = rows_v[r0 + r, pl.ds(co, 16)]
                    m = jnp.maximum(m, t)
                    s = s + t
                    q = q + t * t
                mu_buf[p, pl.ds(co, 16)] = m
                cacc = cacc + s * v_buf[p, pl.ds(co, 16)]
                uacc = uacc + s
                qacc = qacc + q
            cross_v[pl.ds(co, 16)] = cacc
            sumu_v[pl.ds(co, 16)] = uacc
            sumusq_v[pl.ds(co, 16)] = qacc
            return 0

        lax.fori_loop(0, C2 // 16, chunk_body, 0)
        pltpu.sync_copy(mu_buf, mu_hbm.at[pl.ds(w * ppw + g * PB, PB)])

    fire_idx(0, 0)
    fire_idx(1, 1)
    wait_idx(0, 0)
    fire_dat(0, 0)

    def pipe_body(h, _):
        g0 = 2 * h
        g1 = g0 + 1
        wait_idx(g1, 1)
        fire_dat(g1, 1)

        @pl.when(g0 + 2 < nblk)
        def _():
            fire_idx(g0 + 2, 0)

        wait_dat(g0, 0)
        compute(g0, 0)

        @pl.when(g0 + 2 < nblk)
        def _():
            wait_idx(g0 + 2, 0)
            fire_dat(g0 + 2, 0)

        @pl.when(g1 + 2 < nblk)
        def _():
            fire_idx(g1 + 2, 1)

        wait_dat(g1, 1)
        compute(g1, 1)
        return 0

    lax.fori_loop(0, nblk // 2, pipe_body, 0)
    pltpu.sync_copy(cross_v, cross_hbm.at[w])
    pltpu.sync_copy(sumu_v, sumu_hbm.at[w])
    pltpu.sync_copy(sumusq_v, sumusq_hbm.at[w])


def _run_k4(u_flat, v_flat, idx_flat):
    mesh = plsc.VectorSubcoreMesh(core_axis_name="c", subcore_axis_name="s")
    kfn = functools.partial(
        pl.kernel,
        mesh=mesh,
        out_type=[
            jax.ShapeDtypeStruct((B * N, C2), jnp.float32),
            jax.ShapeDtypeStruct((32, C2), jnp.float32),
            jax.ShapeDtypeStruct((32, C2), jnp.float32),
            jax.ShapeDtypeStruct((32, C2), jnp.float32),
        ],
        scratch_types=[
            pltpu.VMEM((EB,), jnp.int32),
            pltpu.VMEM((EB,), jnp.int32),
            pltpu.VMEM((EB,), jnp.int32),
            pltpu.VMEM((EB,), jnp.int32),
            pltpu.VMEM((EB, C2), jnp.float32),
            pltpu.VMEM((EB, C2), jnp.float32),
            pltpu.VMEM((PB, C2), jnp.float32),
            pltpu.VMEM((PB, C2), jnp.float32),
            pltpu.VMEM((PB, C2), jnp.float32),
            pltpu.VMEM((C2,), jnp.float32),
            pltpu.VMEM((C2,), jnp.float32),
            pltpu.VMEM((C2,), jnp.float32),
            pltpu.SemaphoreType.DMA,
            pltpu.SemaphoreType.DMA,
            pltpu.SemaphoreType.DMA,
            pltpu.SemaphoreType.DMA,
        ],
    )(_k4_sc_body)
    return kfn(u_flat, v_flat, idx_flat)


def _k5_body(u_ref, v_ref, mu_ref, sumu_ref, sumusq_ref, cross_ref, fps_ref,
             x_ref, g_ref, bt_ref, coor_ref, newx_ref):
    v = v_ref[0]
    mu = mu_ref[0]
    cross = jnp.sum(cross_ref[0], axis=0, keepdims=True)
    hi = jax.lax.Precision.HIGHEST
    sum_u = jnp.sum(sumu_ref[0], axis=0, keepdims=True)
    sum_usq = jnp.sum(sumusq_ref[0], axis=0, keepdims=True)
    sum_v = jnp.sum(v, axis=0, keepdims=True)
    sum_vsq = jnp.sum(v * v, axis=0, keepdims=True)
    s1 = sum_u + float(K) * sum_v
    s2 = sum_usq + 2.0 * cross + float(K) * sum_vsq
    gsel = (lax.broadcasted_iota(jnp.int32, (C2, NGROUP), 0) // (C2 // NGROUP)
            == lax.broadcasted_iota(jnp.int32, (C2, NGROUP), 1)
            ).astype(jnp.float32)
    s1g = lax.dot_general(s1, gsel, (((1,), (0,)), ((), ())),
                          precision=hi, preferred_element_type=jnp.float32)
    s2g = lax.dot_general(s2, gsel, (((1,), (0,)), ((), ())),
                          precision=hi, preferred_element_type=jnp.float32)
    cnt_total = float((C2 // NGROUP) * N * K)
    mean_g = s1g / cnt_total
    var_g = s2g / cnt_total - mean_g * mean_g
    rstd_g = lax.rsqrt(var_g + 1e-5)
    gselt = (lax.broadcasted_iota(jnp.int32, (NGROUP, C2), 0) ==
             lax.broadcasted_iota(jnp.int32, (NGROUP, C2), 1) // (C2 // NGROUP)
             ).astype(jnp.float32)
    rstd = lax.dot_general(rstd_g, gselt, (((1,), (0,)), ((), ())),
                           precision=hi, preferred_element_type=jnp.float32)
    mean = lax.dot_general(mean_g, gselt, (((1,), (0,)), ((), ())),
                           precision=hi, preferred_element_type=jnp.float32)
    scale = g_ref[...] * rstd
    shift = bt_ref[...] - mean * scale
    act = (mu + v) * scale + shift
    act = jnp.where(act >= 0.0, act, 0.2 * act)
    frow = fps_ref[0]
    pt = (lax.broadcasted_iota(jnp.int32, (N, NPOINT), 0) == frow
          ).astype(jnp.float32)
    newx_ref[0] = lax.dot_general(act, pt, (((0,), (0,)), ((), ())),
                                  precision=hi,
                                  preferred_element_type=jnp.float32)
    xb = x_ref[0]
    coor_ref[0] = lax.dot_general(xb, pt, (((1,), (0,)), ((), ())),
                                  precision=hi,
                                  preferred_element_type=jnp.float32)


def _run_k5(u, v, mu, sumu_part, sumusq_part, cross_part, fps_sm, x, gamma,
            beta):
    return pl.pallas_call(
        _k5_body,
        grid=(B,),
        in_specs=[
            pl.BlockSpec((1, N, C2), lambda b: (b, 0, 0)),
            pl.BlockSpec((1, N, C2), lambda b: (b, 0, 0)),
            pl.BlockSpec((1, N, C2), lambda b: (b, 0, 0)),
            pl.BlockSpec((1, 4, C2), lambda b: (b, 0, 0)),
            pl.BlockSpec((1, 4, C2), lambda b: (b, 0, 0)),
            pl.BlockSpec((1, 4, C2), lambda b: (b, 0, 0)),
            pl.BlockSpec((1, 1, NPOINT), lambda b: (b, 0, 0)),
            pl.BlockSpec((1, 3, N), lambda b: (b, 0, 0)),
            pl.BlockSpec((1, C2), lambda b: (0, 0)),
            pl.BlockSpec((1, C2), lambda b: (0, 0)),
        ],
        out_specs=[
            pl.BlockSpec((1, 3, NPOINT), lambda b: (b, 0, 0)),
            pl.BlockSpec((1, C2, NPOINT), lambda b: (b, 0, 0)),
        ],
        out_shape=[
            jax.ShapeDtypeStruct((B, 3, NPOINT), jnp.float32),
            jax.ShapeDtypeStruct((B, C2, NPOINT), jnp.float32),
        ],
    )(u, v, mu, sumu_part.reshape(B, 4, C2), sumusq_part.reshape(B, 4, C2),
      cross_part.reshape(B, 4, C2), fps_sm.reshape(B, 1, NPOINT), x,
      gamma.reshape(1, C2), beta.reshape(1, C2))


def _k4_jnp_debug(u_flat, v_flat, idx_flat):
    u = u_flat.reshape(B, N, C2)
    v = v_flat.reshape(B, N, C2)
    idx = idx_flat.reshape(B, N, K)
    gath = jax.vmap(lambda uu, ii: uu[ii])(u, idx)
    mu = jnp.max(gath, axis=2)
    su = jnp.sum(gath, axis=2)
    cross_b = jnp.einsum("bnd,bnd->bd", su, v)
    sumu_b = jnp.sum(su, axis=1)
    sumusq_b = jnp.einsum("bnkd->bd", gath * gath)
    cross_part = jnp.zeros((32, C2), jnp.float32).at[0::4].set(cross_b)
    sumu_part = jnp.zeros((32, C2), jnp.float32).at[0::4].set(sumu_b)
    sumusq_part = jnp.zeros((32, C2), jnp.float32).at[0::4].set(sumusq_b)
    return mu.reshape(B * N, C2), cross_part, sumu_part, sumusq_part


def kernel(x, f, W1, b1, W2, gamma, beta):
    xt = jnp.swapaxes(x, 1, 2)
    u, v = _run_k1(f, W1, b1, W2)
    idx = _run_k2(x, xt)
    fps_sm = _run_k3(x)
    mu_flat, cross_part, sumu_part, sumusq_part = _run_k4(
        u.reshape(B * N, C2), v.reshape(B * N, C2),
        idx.reshape(B * N * K))
    mu = mu_flat.reshape(B, N, C2)
    new_coor, new_x = _run_k5(u, v, mu, sumu_part, sumusq_part, cross_part,
                              fps_sm, x, gamma, beta)
    return (new_coor, new_x, fps_sm)


# speedup vs baseline: 20.7061x; 1.0208x over previous
import functools

import jax
import jax.numpy as jnp
from jax import lax
from jax.experimental import pallas as pl
from jax.experimental.pallas import tpu as pltpu
from jax.experimental.pallas import tpu_sc as plsc

B = 8
N = 2048
K = 16
C1 = 64
C2 = 256
NGROUP = 4
NPOINT = 512
ROWS = 256


def _k1_body(f_ref, w1_ref, b1_ref, w2_ref, u_ref, v_ref):
    fb = f_ref[0]
    f64 = lax.dot_general(fb, w1_ref[...], (((0,), (0,)), ((), ())),
                          preferred_element_type=jnp.float32)
    f64 = f64 + b1_ref[...]
    w2a = w2_ref[0:C1, :]
    w2b = w2_ref[C1:2 * C1, :]
    u_ref[0] = lax.dot_general(f64, w2a, (((1,), (0,)), ((), ())),
                               preferred_element_type=jnp.float32)
    v_ref[0] = lax.dot_general(f64, w2b - w2a, (((1,), (0,)), ((), ())),
                               preferred_element_type=jnp.float32)


def _run_k1(f, W1, b1, W2):
    return pl.pallas_call(
        _k1_body,
        grid=(B,),
        in_specs=[
            pl.BlockSpec((1, 3, N), lambda b: (b, 0, 0)),
            pl.BlockSpec((3, C1), lambda b: (0, 0)),
            pl.BlockSpec((1, C1), lambda b: (0, 0)),
            pl.BlockSpec((2 * C1, C2), lambda b: (0, 0)),
        ],
        out_specs=[
            pl.BlockSpec((1, N, C2), lambda b: (b, 0, 0)),
            pl.BlockSpec((1, N, C2), lambda b: (b, 0, 0)),
        ],
        out_shape=[
            jax.ShapeDtypeStruct((B, N, C2), jnp.float32),
            jax.ShapeDtypeStruct((B, N, C2), jnp.float32),
        ],
    )(f, W1, b1.reshape(1, C1), W2)


def _k2_body(xr_ref, x_ref, idx_ref):
    xr = xr_ref[0]
    xb = x_ref[0]
    g = lax.dot_general(xr, xb, (((1,), (0,)), ((), ())),
                        preferred_element_type=jnp.float32)
    xi2 = jnp.sum(xr * xr, axis=1, keepdims=True)
    xj2 = jnp.sum(xb * xb, axis=0, keepdims=True)
    cur = (-2.0 * g + xi2) + xj2
    iota_f = lax.broadcasted_iota(jnp.int32, (ROWS, N), 1).astype(jnp.float32)
    cols = []
    for _ in range(K):
        m = jnp.min(cur, axis=1, keepdims=True)
        cand = jnp.where(cur == m, iota_f, float(N))
        sel = jnp.min(cand, axis=1, keepdims=True)
        cols.append(sel.astype(jnp.int32))
        cur = jnp.where(iota_f == sel, jnp.inf, cur)
    idx_ref[0] = jnp.concatenate(cols, axis=1)


def _run_k2(x, xt):
    return pl.pallas_call(
        _k2_body,
        grid=(B, N // ROWS),
        in_specs=[
            pl.BlockSpec((1, ROWS, 3), lambda b, r: (b, r, 0)),
            pl.BlockSpec((1, 3, N), lambda b, r: (b, 0, 0)),
        ],
        out_specs=pl.BlockSpec((1, ROWS, K), lambda b, r: (b, r, 0)),
        out_shape=jax.ShapeDtypeStruct((B, N, K), jnp.int32),
    )(xt, x)


def _k3_body(x_ref, out_ref):
    x0 = x_ref[:, 0, :]
    x1 = x_ref[:, 1, :]
    x2 = x_ref[:, 2, :]
    x012 = jnp.concatenate([x0, x1, x2], axis=0)
    iota_f = lax.broadcasted_iota(jnp.int32, (B, N), 1).astype(jnp.float32)
    iota3_f = lax.broadcasted_iota(jnp.int32, (3 * B, N), 1
                                   ).astype(jnp.float32)
    iota_np = lax.broadcasted_iota(jnp.int32, (B, NPOINT), 1)

    def body(i, carry):
        dist, far, fars = carry
        far_mat = far.astype(jnp.int32) + 0 * iota_np
        fars = jnp.where(iota_np == i, far_mat, fars)
        far3 = jnp.concatenate([far, far, far], axis=0)
        oh3 = iota3_f == far3
        cen = jnp.sum(jnp.where(oh3, x012, 0.0), axis=1, keepdims=True)
        cx = cen[0:B]
        cy = cen[B:2 * B]
        cz = cen[2 * B:3 * B]
        dx = x0 - cx
        dy = x1 - cy
        dz = x2 - cz
        d = (dx * dx + dy * dy) + dz * dz
        dist = jnp.minimum(dist, d)
        m = jnp.max(dist, axis=1, keepdims=True)
        cand = jnp.where(dist == m, iota_f, float(N))
        far_new = jnp.min(cand, axis=1, keepdims=True)
        return dist, far_new, fars

    init = (1e10 + 0.0 * x0,
            0.0 * jnp.min(x0, axis=1, keepdims=True),
            (0.0 * x0[:, :NPOINT]).astype(jnp.int32))
    _, _, fars = lax.fori_loop(0, NPOINT, body, init)
    out_ref[...] = fars


def _run_k3(x):
    return pl.pallas_call(
        _k3_body,
        grid=(1,),
        in_specs=[pl.BlockSpec((B, 3, N), lambda i: (0, 0, 0))],
        out_specs=pl.BlockSpec((B, NPOINT), lambda i: (0, 0)),
        out_shape=jax.ShapeDtypeStruct((B, NPOINT), jnp.int32),
    )(x)


PB = 8
EB = PB * K


def _k4_sc_body(u_hbm, v_hbm, idx_hbm, mu_hbm, cross_hbm, sumu_hbm,
                sumusq_hbm, idx0, idx1, idxg0, idxg1, rows0, rows1, vb0, vb1,
                mu_buf, cross_v, sumu_v, sumusq_v, si0, si1, sd0, sd1):
    info = plsc.get_sparse_core_info()
    nc = info.num_cores
    w = lax.axis_index("s") * nc + lax.axis_index("c")
    ppw = (B * N) // (nc * info.num_subcores)
    base = (w // (N // ppw)) * N
    nblk = ppw // PB
    bufs = [(idx0, idxg0, rows0, vb0, si0, sd0),
            (idx1, idxg1, rows1, vb1, si1, sd1)]

    zero16f = jnp.zeros((16,), jnp.float32)
    for j in range(C2 // 16):
        cross_v[pl.ds(j * 16, 16)] = zero16f
        sumu_v[pl.ds(j * 16, 16)] = zero16f
        sumusq_v[pl.ds(j * 16, 16)] = zero16f

    def fire_idx(g, k):
        idx_v, _, _, _, si, _ = bufs[k]
        pltpu.async_copy(idx_hbm.at[pl.ds((w * ppw + g * PB) * K, EB)],
                         idx_v, si)

    def wait_idx(g, k):
        idx_v, _, _, _, si, _ = bufs[k]
        pltpu.make_async_copy(idx_hbm.at[pl.ds((w * ppw + g * PB) * K, EB)],
                              idx_v, si).wait()

    def fire_dat(g, k):
        idx_v, idxg_v, rows_v, v_buf, _, sd = bufs[k]
        for j in range(EB // 16):
            t = idx_v[pl.ds(j * 16, 16)]
            idxg_v[pl.ds(j * 16, 16)] = t + base
        pltpu.async_copy(u_hbm.at[idxg_v], rows_v, sd)
        pltpu.async_copy(v_hbm.at[pl.ds(w * ppw + g * PB, PB)], v_buf, sd)

    def wait_dat(g, k):
        _, idxg_v, rows_v, v_buf, _, sd = bufs[k]
        pltpu.make_async_copy(u_hbm.at[idxg_v], rows_v, sd).wait()
        pltpu.make_async_copy(v_hbm.at[pl.ds(w * ppw + g * PB, PB)], v_buf,
                              sd).wait()

    def compute(g, k):
        _, _, rows_v, v_buf, _, _ = bufs[k]

        def chunk_body(c, _):
            co = c * 16
            cacc = cross_v[pl.ds(co, 16)]
            uacc = sumu_v[pl.ds(co, 16)]
            qacc = sumusq_v[pl.ds(co, 16)]
            for p in range(PB):
                r0 = p * K
                m = rows_v[r0, pl.ds(co, 16)]
                s = m
                q = m * m
                for r in range(1, K):
                    t = rows_v[r0 + r, pl.ds(co, 16)]
                    m = jnp.maximum(m, t)
                    s = s + t
                    q = q + t * t
                mu_buf[p, pl.ds(co, 16)] = m
                cacc = cacc + s * v_buf[p, pl.ds(co, 16)]
                uacc = uacc + s
                qacc = qacc + q
            cross_v[pl.ds(co, 16)] = cacc
            sumu_v[pl.ds(co, 16)] = uacc
            sumusq_v[pl.ds(co, 16)] = qacc
            return 0

        lax.fori_loop(0, C2 // 16, chunk_body, 0)
        pltpu.sync_copy(mu_buf, mu_hbm.at[pl.ds(w * ppw + g * PB, PB)])

    fire_idx(0, 0)
    fire_idx(1, 1)
    wait_idx(0, 0)
    fire_dat(0, 0)

    def pipe_body(h, _):
        g0 = 2 * h
        g1 = g0 + 1
        wait_idx(g1, 1)
        fire_dat(g1, 1)

        @pl.when(g0 + 2 < nblk)
        def _():
            fire_idx(g0 + 2, 0)

        wait_dat(g0, 0)
        compute(g0, 0)

        @pl.when(g0 + 2 < nblk)
        def _():
            wait_idx(g0 + 2, 0)
            fire_dat(g0 + 2, 0)

        @pl.when(g1 + 2 < nblk)
        def _():
            fire_idx(g1 + 2, 1)

        wait_dat(g1, 1)
        compute(g1, 1)
        return 0

    lax.fori_loop(0, nblk // 2, pipe_body, 0)
    pltpu.sync_copy(cross_v, cross_hbm.at[w])
    pltpu.sync_copy(sumu_v, sumu_hbm.at[w])
    pltpu.sync_copy(sumusq_v, sumusq_hbm.at[w])


def _run_k4(u_flat, v_flat, idx_flat):
    mesh = plsc.VectorSubcoreMesh(core_axis_name="c", subcore_axis_name="s")
    kfn = functools.partial(
        pl.kernel,
        mesh=mesh,
        out_type=[
            jax.ShapeDtypeStruct((B * N, C2), jnp.float32),
            jax.ShapeDtypeStruct((32, C2), jnp.float32),
            jax.ShapeDtypeStruct((32, C2), jnp.float32),
            jax.ShapeDtypeStruct((32, C2), jnp.float32),
        ],
        scratch_types=[
            pltpu.VMEM((EB,), jnp.int32),
            pltpu.VMEM((EB,), jnp.int32),
            pltpu.VMEM((EB,), jnp.int32),
            pltpu.VMEM((EB,), jnp.int32),
            pltpu.VMEM((EB, C2), jnp.float32),
            pltpu.VMEM((EB, C2), jnp.float32),
            pltpu.VMEM((PB, C2), jnp.float32),
            pltpu.VMEM((PB, C2), jnp.float32),
            pltpu.VMEM((PB, C2), jnp.float32),
            pltpu.VMEM((C2,), jnp.float32),
            pltpu.VMEM((C2,), jnp.float32),
            pltpu.VMEM((C2,), jnp.float32),
            pltpu.SemaphoreType.DMA,
            pltpu.SemaphoreType.DMA,
            pltpu.SemaphoreType.DMA,
            pltpu.SemaphoreType.DMA,
        ],
    )(_k4_sc_body)
    return kfn(u_flat, v_flat, idx_flat)


def _k5_body(u_ref, v_ref, mu_ref, sumu_ref, sumusq_ref, cross_ref, fps_ref,
             x_ref, g_ref, bt_ref, coor_ref, newx_ref):
    v = v_ref[0]
    mu = mu_ref[0]
    cross = jnp.sum(cross_ref[0], axis=0, keepdims=True)
    hi = jax.lax.Precision.HIGHEST
    sum_u = jnp.sum(sumu_ref[0], axis=0, keepdims=True)
    sum_usq = jnp.sum(sumusq_ref[0], axis=0, keepdims=True)
    sum_v = jnp.sum(v, axis=0, keepdims=True)
    sum_vsq = jnp.sum(v * v, axis=0, keepdims=True)
    s1 = sum_u + float(K) * sum_v
    s2 = sum_usq + 2.0 * cross + float(K) * sum_vsq
    gsel = (lax.broadcasted_iota(jnp.int32, (C2, NGROUP), 0) // (C2 // NGROUP)
            == lax.broadcasted_iota(jnp.int32, (C2, NGROUP), 1)
            ).astype(jnp.float32)
    s1g = lax.dot_general(s1, gsel, (((1,), (0,)), ((), ())),
                          precision=hi, preferred_element_type=jnp.float32)
    s2g = lax.dot_general(s2, gsel, (((1,), (0,)), ((), ())),
                          precision=hi, preferred_element_type=jnp.float32)
    cnt_total = float((C2 // NGROUP) * N * K)
    mean_g = s1g / cnt_total
    var_g = s2g / cnt_total - mean_g * mean_g
    rstd_g = lax.rsqrt(var_g + 1e-5)
    gselt = (lax.broadcasted_iota(jnp.int32, (NGROUP, C2), 0) ==
             lax.broadcasted_iota(jnp.int32, (NGROUP, C2), 1) // (C2 // NGROUP)
             ).astype(jnp.float32)
    rstd = lax.dot_general(rstd_g, gselt, (((1,), (0,)), ((), ())),
                           precision=hi, preferred_element_type=jnp.float32)
    mean = lax.dot_general(mean_g, gselt, (((1,), (0,)), ((), ())),
                           precision=hi, preferred_element_type=jnp.float32)
    scale = g_ref[...] * rstd
    shift = bt_ref[...] - mean * scale
    act = (mu + v) * scale + shift
    act = jnp.where(act >= 0.0, act, 0.2 * act)
    frow = fps_ref[0]
    pt = (lax.broadcasted_iota(jnp.int32, (N, NPOINT), 0) == frow
          ).astype(jnp.float32)
    newx_ref[0] = lax.dot_general(act, pt, (((0,), (0,)), ((), ())),
                                  precision=hi,
                                  preferred_element_type=jnp.float32)
    xb = x_ref[0]
    coor_ref[0] = lax.dot_general(xb, pt, (((1,), (0,)), ((), ())),
                                  precision=hi,
                                  preferred_element_type=jnp.float32)


def _run_k5(u, v, mu, sumu_part, sumusq_part, cross_part, fps_sm, x, gamma,
            beta):
    return pl.pallas_call(
        _k5_body,
        grid=(B,),
        in_specs=[
            pl.BlockSpec((1, N, C2), lambda b: (b, 0, 0)),
            pl.BlockSpec((1, N, C2), lambda b: (b, 0, 0)),
            pl.BlockSpec((1, N, C2), lambda b: (b, 0, 0)),
            pl.BlockSpec((1, 4, C2), lambda b: (b, 0, 0)),
            pl.BlockSpec((1, 4, C2), lambda b: (b, 0, 0)),
            pl.BlockSpec((1, 4, C2), lambda b: (b, 0, 0)),
            pl.BlockSpec((1, 1, NPOINT), lambda b: (b, 0, 0)),
            pl.BlockSpec((1, 3, N), lambda b: (b, 0, 0)),
            pl.BlockSpec((1, C2), lambda b: (0, 0)),
            pl.BlockSpec((1, C2), lambda b: (0, 0)),
        ],
        out_specs=[
            pl.BlockSpec((1, 3, NPOINT), lambda b: (b, 0, 0)),
            pl.BlockSpec((1, C2, NPOINT), lambda b: (b, 0, 0)),
        ],
        out_shape=[
            jax.ShapeDtypeStruct((B, 3, NPOINT), jnp.float32),
            jax.ShapeDtypeStruct((B, C2, NPOINT), jnp.float32),
        ],
    )(u, v, mu, sumu_part.reshape(B, 4, C2), sumusq_part.reshape(B, 4, C2),
      cross_part.reshape(B, 4, C2), fps_sm.reshape(B, 1, NPOINT), x,
      gamma.reshape(1, C2), beta.reshape(1, C2))


def _k4_jnp_debug(u_flat, v_flat, idx_flat):
    u = u_flat.reshape(B, N, C2)
    v = v_flat.reshape(B, N, C2)
    idx = idx_flat.reshape(B, N, K)
    gath = jax.vmap(lambda uu, ii: uu[ii])(u, idx)
    mu = jnp.max(gath, axis=2)
    su = jnp.sum(gath, axis=2)
    cross_b = jnp.einsum("bnd,bnd->bd", su, v)
    sumu_b = jnp.sum(su, axis=1)
    sumusq_b = jnp.einsum("bnkd->bd", gath * gath)
    cross_part = jnp.zeros((32, C2), jnp.float32).at[0::4].set(cross_b)
    sumu_part = jnp.zeros((32, C2), jnp.float32).at[0::4].set(sumu_b)
    sumusq_part = jnp.zeros((32, C2), jnp.float32).at[0::4].set(sumusq_b)
    return mu.reshape(B * N, C2), cross_part, sumu_part, sumusq_part


def kernel(x, f, W1, b1, W2, gamma, beta):
    xt = jnp.swapaxes(x, 1, 2)
    u, v = _run_k1(f, W1, b1, W2)
    idx = _run_k2(x, xt)
    fps_sm = _run_k3(x)
    mu_flat, cross_part, sumu_part, sumusq_part = _run_k4(
        u.reshape(B * N, C2), v.reshape(B * N, C2),
        idx.reshape(B * N * K))
    mu = mu_flat.reshape(B, N, C2)
    new_coor, new_x = _run_k5(u, v, mu, sumu_part, sumusq_part, cross_part,
                              fps_sm, x, gamma, beta)
    return (new_coor, new_x, fps_sm)


# speedup vs baseline: 20.7230x; 1.0008x over previous
import functools

import jax
import jax.numpy as jnp
from jax import lax
from jax.experimental import pallas as pl
from jax.experimental.pallas import tpu as pltpu
from jax.experimental.pallas import tpu_sc as plsc

B = 8
N = 2048
K = 16
C1 = 64
C2 = 256
NGROUP = 4
NPOINT = 512
ROWS = 256


def _k1_body(f_ref, w1_ref, b1_ref, w2_ref, u_ref, v_ref):
    fb = f_ref[0]
    f64 = lax.dot_general(fb, w1_ref[...], (((0,), (0,)), ((), ())),
                          preferred_element_type=jnp.float32)
    f64 = f64 + b1_ref[...]
    w2a = w2_ref[0:C1, :]
    w2b = w2_ref[C1:2 * C1, :]
    u_ref[0] = lax.dot_general(f64, w2a, (((1,), (0,)), ((), ())),
                               preferred_element_type=jnp.float32)
    v_ref[0] = lax.dot_general(f64, w2b - w2a, (((1,), (0,)), ((), ())),
                               preferred_element_type=jnp.float32)


def _run_k1(f, W1, b1, W2):
    return pl.pallas_call(
        _k1_body,
        grid=(B,),
        in_specs=[
            pl.BlockSpec((1, 3, N), lambda b: (b, 0, 0)),
            pl.BlockSpec((3, C1), lambda b: (0, 0)),
            pl.BlockSpec((1, C1), lambda b: (0, 0)),
            pl.BlockSpec((2 * C1, C2), lambda b: (0, 0)),
        ],
        out_specs=[
            pl.BlockSpec((1, N, C2), lambda b: (b, 0, 0)),
            pl.BlockSpec((1, N, C2), lambda b: (b, 0, 0)),
        ],
        out_shape=[
            jax.ShapeDtypeStruct((B, N, C2), jnp.float32),
            jax.ShapeDtypeStruct((B, N, C2), jnp.float32),
        ],
    )(f, W1, b1.reshape(1, C1), W2)


def _k2_body(xr_ref, x_ref, idx_ref):
    xr = xr_ref[0]
    xb = x_ref[0]
    g = lax.dot_general(xr, xb, (((1,), (0,)), ((), ())),
                        preferred_element_type=jnp.float32)
    xi2 = jnp.sum(xr * xr, axis=1, keepdims=True)
    xj2 = jnp.sum(xb * xb, axis=0, keepdims=True)
    cur = (-2.0 * g + xi2) + xj2
    iota_f = lax.broadcasted_iota(jnp.int32, (ROWS, N), 1).astype(jnp.float32)
    cols = []
    for _ in range(K):
        m = jnp.min(cur, axis=1, keepdims=True)
        cand = jnp.where(cur == m, iota_f, float(N))
        sel = jnp.min(cand, axis=1, keepdims=True)
        cols.append(sel.astype(jnp.int32))
        cur = jnp.where(iota_f == sel, jnp.inf, cur)
    idx_ref[0] = jnp.concatenate(cols, axis=1)


def _run_k2(x, xt):
    return pl.pallas_call(
        _k2_body,
        grid=(B, N // ROWS),
        in_specs=[
            pl.BlockSpec((1, ROWS, 3), lambda b, r: (b, r, 0)),
            pl.BlockSpec((1, 3, N), lambda b, r: (b, 0, 0)),
        ],
        out_specs=pl.BlockSpec((1, ROWS, K), lambda b, r: (b, r, 0)),
        out_shape=jax.ShapeDtypeStruct((B, N, K), jnp.int32),
    )(xt, x)


def _k3_body(x_ref, out_ref):
    x0 = x_ref[:, 0, :]
    x1 = x_ref[:, 1, :]
    x2 = x_ref[:, 2, :]
    x012 = jnp.concatenate([x0, x1, x2], axis=0)
    iota_f = lax.broadcasted_iota(jnp.int32, (B, N), 1).astype(jnp.float32)
    iota3_f = lax.broadcasted_iota(jnp.int32, (3 * B, N), 1
                                   ).astype(jnp.float32)
    iota_np = lax.broadcasted_iota(jnp.int32, (B, NPOINT), 1)

    def body(i, carry):
        dist, far, fars = carry
        far_mat = far.astype(jnp.int32) + 0 * iota_np
        fars = jnp.where(iota_np == i, far_mat, fars)
        far3 = jnp.concatenate([far, far, far], axis=0)
        oh3 = iota3_f == far3
        cen = jnp.sum(jnp.where(oh3, x012, 0.0), axis=1, keepdims=True)
        cx = cen[0:B]
        cy = cen[B:2 * B]
        cz = cen[2 * B:3 * B]
        dx = x0 - cx
        dy = x1 - cy
        dz = x2 - cz
        d = (dx * dx + dy * dy) + dz * dz
        dist = jnp.minimum(dist, d)
        m = jnp.max(dist, axis=1, keepdims=True)
        cand = jnp.where(dist == m, iota_f, float(N))
        far_new = jnp.min(cand, axis=1, keepdims=True)
        return dist, far_new, fars

    init = (1e10 + 0.0 * x0,
            0.0 * jnp.min(x0, axis=1, keepdims=True),
            (0.0 * x0[:, :NPOINT]).astype(jnp.int32))
    _, _, fars = lax.fori_loop(0, NPOINT, body, init)
    out_ref[...] = fars


def _run_k3(x):
    return pl.pallas_call(
        _k3_body,
        grid=(1,),
        in_specs=[pl.BlockSpec((B, 3, N), lambda i: (0, 0, 0))],
        out_specs=pl.BlockSpec((B, NPOINT), lambda i: (0, 0)),
        out_shape=jax.ShapeDtypeStruct((B, NPOINT), jnp.int32),
    )(x)


PB = 8
EB = PB * K


def _k4_sc_body(u_hbm, v_hbm, idx_hbm, mu_hbm, cross_hbm, sumu_hbm,
                sumusq_hbm, idx0, idx1, idxg0, idxg1, rows0, rows1, vb0, vb1,
                mu_buf, cross_v, sumu_v, sumusq_v, si0, si1, sd0, sd1):
    info = plsc.get_sparse_core_info()
    nc = info.num_cores
    w = lax.axis_index("s") * nc + lax.axis_index("c")
    ppw = (B * N) // (nc * info.num_subcores)
    base = (w // (N // ppw)) * N
    nblk = ppw // PB
    bufs = [(idx0, idxg0, rows0, vb0, si0, sd0),
            (idx1, idxg1, rows1, vb1, si1, sd1)]

    zero16f = jnp.zeros((16,), jnp.float32)
    for j in range(C2 // 16):
        cross_v[pl.ds(j * 16, 16)] = zero16f
        sumu_v[pl.ds(j * 16, 16)] = zero16f
        sumusq_v[pl.ds(j * 16, 16)] = zero16f

    def fire_idx(g, k):
        idx_v, _, _, _, si, _ = bufs[k]
        pltpu.async_copy(idx_hbm.at[pl.ds((w * ppw + g * PB) * K, EB)],
                         idx_v, si)

    def wait_idx(g, k):
        idx_v, _, _, _, si, _ = bufs[k]
        pltpu.make_async_copy(idx_hbm.at[pl.ds((w * ppw + g * PB) * K, EB)],
                              idx_v, si).wait()

    def fire_dat(g, k):
        idx_v, idxg_v, rows_v, v_buf, _, sd = bufs[k]
        for j in range(EB // 16):
            t = idx_v[pl.ds(j * 16, 16)]
            idxg_v[pl.ds(j * 16, 16)] = t + base
        pltpu.async_copy(u_hbm.at[idxg_v], rows_v, sd)
        pltpu.async_copy(v_hbm.at[pl.ds(w * ppw + g * PB, PB)], v_buf, sd)

    def wait_dat(g, k):
        _, idxg_v, rows_v, v_buf, _, sd = bufs[k]
        pltpu.make_async_copy(u_hbm.at[idxg_v], rows_v, sd).wait()
        pltpu.make_async_copy(v_hbm.at[pl.ds(w * ppw + g * PB, PB)], v_buf,
                              sd).wait()

    def compute(g, k):
        _, _, rows_v, v_buf, _, _ = bufs[k]

        def chunk_body(c, _):
            co = c * 16
            cacc = cross_v[pl.ds(co, 16)]
            uacc = sumu_v[pl.ds(co, 16)]
            qacc = sumusq_v[pl.ds(co, 16)]
            for p in range(PB):
                r0 = p * K
                m = rows_v[r0, pl.ds(co, 16)]
                s = m
                q = m * m
                for r in range(1, K):
                    t = rows_v[r0 + r, pl.ds(co, 16)]
                    m = jnp.maximum(m, t)
                    s = s + t
                    q = q + t * t
                mu_buf[p, pl.ds(co, 16)] = m
                cacc = cacc + s * v_buf[p, pl.ds(co, 16)]
                uacc = uacc + s
                qacc = qacc + q
            cross_v[pl.ds(co, 16)] = cacc
            sumu_v[pl.ds(co, 16)] = uacc
            sumusq_v[pl.ds(co, 16)] = qacc
            return 0

        lax.fori_loop(0, C2 // 16, chunk_body, 0)
        pltpu.sync_copy(mu_buf, mu_hbm.at[pl.ds(w * ppw + g * PB, PB)])

    fire_idx(0, 0)
    fire_idx(1, 1)
    wait_idx(0, 0)
    fire_dat(0, 0)

    def pipe_body(h, _):
        g0 = 2 * h
        g1 = g0 + 1
        wait_idx(g1, 1)
        fire_dat(g1, 1)

        @pl.when(g0 + 2 < nblk)
        def _():
            fire_idx(g0 + 2, 0)

        wait_dat(g0, 0)
        compute(g0, 0)

        @pl.when(g0 + 2 < nblk)
        def _():
            wait_idx(g0 + 2, 0)
            fire_dat(g0 + 2, 0)

        @pl.when(g1 + 2 < nblk)
        def _():
            fire_idx(g1 + 2, 1)

        wait_dat(g1, 1)
        compute(g1, 1)
        return 0

    lax.fori_loop(0, nblk // 2, pipe_body, 0)
    pltpu.sync_copy(cross_v, cross_hbm.at[w])
    pltpu.sync_copy(sumu_v, sumu_hbm.at[w])
    pltpu.sync_copy(sumusq_v, sumusq_hbm.at[w])


def _run_k4(u_flat, v_flat, idx_flat):
    mesh = plsc.VectorSubcoreMesh(core_axis_name="c", subcore_axis_name="s")
    kfn = functools.partial(
        pl.kernel,
        mesh=mesh,
        out_type=[
            jax.ShapeDtypeStruct((B * N, C2), jnp.float32),
            jax.ShapeDtypeStruct((32, C2), jnp.float32),
            jax.ShapeDtypeStruct((32, C2), jnp.float32),
            jax.ShapeDtypeStruct((32, C2), jnp.float32),
        ],
        scratch_types=[
            pltpu.VMEM((EB,), jnp.int32),
            pltpu.VMEM((EB,), jnp.int32),
            pltpu.VMEM((EB,), jnp.int32),
            pltpu.VMEM((EB,), jnp.int32),
            pltpu.VMEM((EB, C2), jnp.float32),
            pltpu.VMEM((EB, C2), jnp.float32),
            pltpu.VMEM((PB, C2), jnp.float32),
            pltpu.VMEM((PB, C2), jnp.float32),
            pltpu.VMEM((PB, C2), jnp.float32),
            pltpu.VMEM((C2,), jnp.float32),
            pltpu.VMEM((C2,), jnp.float32),
            pltpu.VMEM((C2,), jnp.float32),
            pltpu.SemaphoreType.DMA,
            pltpu.SemaphoreType.DMA,
            pltpu.SemaphoreType.DMA,
            pltpu.SemaphoreType.DMA,
        ],
    )(_k4_sc_body)
    return kfn(u_flat, v_flat, idx_flat)


def _k5_body(u_ref, v_ref, mu_ref, sumu_ref, sumusq_ref, cross_ref, fps_ref,
             x_ref, g_ref, bt_ref, coor_ref, newx_ref):
    v = v_ref[0]
    mu = mu_ref[0]
    cross = jnp.sum(cross_ref[0], axis=0, keepdims=True)
    hi = jax.lax.Precision.HIGHEST
    sum_u = jnp.sum(sumu_ref[0], axis=0, keepdims=True)
    sum_usq = jnp.sum(sumusq_ref[0], axis=0, keepdims=True)
    sum_v = jnp.sum(v, axis=0, keepdims=True)
    sum_vsq = jnp.sum(v * v, axis=0, keepdims=True)
    s1 = sum_u + float(K) * sum_v
    s2 = sum_usq + 2.0 * cross + float(K) * sum_vsq
    gsel = (lax.broadcasted_iota(jnp.int32, (C2, NGROUP), 0) // (C2 // NGROUP)
            == lax.broadcasted_iota(jnp.int32, (C2, NGROUP), 1)
            ).astype(jnp.float32)
    s1g = lax.dot_general(s1, gsel, (((1,), (0,)), ((), ())),
                          precision=hi, preferred_element_type=jnp.float32)
    s2g = lax.dot_general(s2, gsel, (((1,), (0,)), ((), ())),
                          precision=hi, preferred_element_type=jnp.float32)
    cnt_total = float((C2 // NGROUP) * N * K)
    mean_g = s1g / cnt_total
    var_g = s2g / cnt_total - mean_g * mean_g
    rstd_g = lax.rsqrt(var_g + 1e-5)
    gselt = (lax.broadcasted_iota(jnp.int32, (NGROUP, C2), 0) ==
             lax.broadcasted_iota(jnp.int32, (NGROUP, C2), 1) // (C2 // NGROUP)
             ).astype(jnp.float32)
    rstd = lax.dot_general(rstd_g, gselt, (((1,), (0,)), ((), ())),
                           precision=hi, preferred_element_type=jnp.float32)
    mean = lax.dot_general(mean_g, gselt, (((1,), (0,)), ((), ())),
                           precision=hi, preferred_element_type=jnp.float32)
    scale = g_ref[...] * rstd
    shift = bt_ref[...] - mean * scale
    act = (mu + v) * scale + shift
    act = jnp.where(act >= 0.0, act, 0.2 * act)
    frow = fps_ref[0]
    pt = (lax.broadcasted_iota(jnp.int32, (N, NPOINT), 0) == frow
          ).astype(jnp.float32)
    newx_ref[0] = lax.dot_general(act, pt, (((0,), (0,)), ((), ())),
                                  precision=hi,
                                  preferred_element_type=jnp.float32)
    xb = x_ref[0]
    coor_ref[0] = lax.dot_general(xb, pt, (((1,), (0,)), ((), ())),
                                  precision=hi,
                                  preferred_element_type=jnp.float32)


def _run_k5(u, v, mu, sumu_part, sumusq_part, cross_part, fps_sm, x, gamma,
            beta):
    return pl.pallas_call(
        _k5_body,
        grid=(B,),
        in_specs=[
            pl.BlockSpec((1, N, C2), lambda b: (b, 0, 0)),
            pl.BlockSpec((1, N, C2), lambda b: (b, 0, 0)),
            pl.BlockSpec((1, N, C2), lambda b: (b, 0, 0)),
            pl.BlockSpec((1, 4, C2), lambda b: (b, 0, 0)),
            pl.BlockSpec((1, 4, C2), lambda b: (b, 0, 0)),
            pl.BlockSpec((1, 4, C2), lambda b: (b, 0, 0)),
            pl.BlockSpec((1, 1, NPOINT), lambda b: (b, 0, 0)),
            pl.BlockSpec((1, 3, N), lambda b: (b, 0, 0)),
            pl.BlockSpec((1, C2), lambda b: (0, 0)),
            pl.BlockSpec((1, C2), lambda b: (0, 0)),
        ],
        out_specs=[
            pl.BlockSpec((1, 3, NPOINT), lambda b: (b, 0, 0)),
            pl.BlockSpec((1, C2, NPOINT), lambda b: (b, 0, 0)),
        ],
        out_shape=[
            jax.ShapeDtypeStruct((B, 3, NPOINT), jnp.float32),
            jax.ShapeDtypeStruct((B, C2, NPOINT), jnp.float32),
        ],
    )(u, v, mu, sumu_part.reshape(B, 4, C2), sumusq_part.reshape(B, 4, C2),
      cross_part.reshape(B, 4, C2), fps_sm.reshape(B, 1, NPOINT), x,
      gamma.reshape(1, C2), beta.reshape(1, C2))


def _k4_jnp_debug(u_flat, v_flat, idx_flat):
    u = u_flat.reshape(B, N, C2)
    v = v_flat.reshape(B, N, C2)
    idx = idx_flat.reshape(B, N, K)
    gath = jax.vmap(lambda uu, ii: uu[ii])(u, idx)
    mu = jnp.max(gath, axis=2)
    su = jnp.sum(gath, axis=2)
    cross_b = jnp.einsum("bnd,bnd->bd", su, v)
    sumu_b = jnp.sum(su, axis=1)
    sumusq_b = jnp.einsum("bnkd->bd", gath * gath)
    cross_part = jnp.zeros((32, C2), jnp.float32).at[0::4].set(cross_b)
    sumu_part = jnp.zeros((32, C2), jnp.float32).at[0::4].set(sumu_b)
    sumusq_part = jnp.zeros((32, C2), jnp.float32).at[0::4].set(sumusq_b)
    return mu.reshape(B * N, C2), cross_part, sumu_part, sumusq_part


def kernel(x, f, W1, b1, W2, gamma, beta):
    xt = jnp.swapaxes(x, 1, 2)
    u, v = _run_k1(f, W1, b1, W2)
    idx = _run_k2(x, xt)
    mu_flat, cross_part, sumu_part, sumusq_part = _run_k4(
        u.reshape(B * N, C2), v.reshape(B * N, C2),
        idx.reshape(B * N * K))
    fps_sm = _run_k3(x)
    mu = mu_flat.reshape(B, N, C2)
    new_coor, new_x = _run_k5(u, v, mu, sumu_part, sumusq_part, cross_part,
                              fps_sm, x, gamma, beta)
    return (new_coor, new_x, fps_sm)


# speedup vs baseline: 20.9169x; 1.0094x over previous
import functools

import jax
import jax.numpy as jnp
from jax import lax
from jax.experimental import pallas as pl
from jax.experimental.pallas import tpu as pltpu
from jax.experimental.pallas import tpu_sc as plsc

B = 8
N = 2048
K = 16
C1 = 64
C2 = 256
NGROUP = 4
NPOINT = 512
ROWS = 512


def _k1_body(f_ref, w1_ref, b1_ref, w2_ref, u_ref, v_ref):
    fb = f_ref[0]
    f64 = lax.dot_general(fb, w1_ref[...], (((0,), (0,)), ((), ())),
                          preferred_element_type=jnp.float32)
    f64 = f64 + b1_ref[...]
    w2a = w2_ref[0:C1, :]
    w2b = w2_ref[C1:2 * C1, :]
    u_ref[0] = lax.dot_general(f64, w2a, (((1,), (0,)), ((), ())),
                               preferred_element_type=jnp.float32)
    v_ref[0] = lax.dot_general(f64, w2b - w2a, (((1,), (0,)), ((), ())),
                               preferred_element_type=jnp.float32)


def _run_k1(f, W1, b1, W2):
    return pl.pallas_call(
        _k1_body,
        grid=(B,),
        in_specs=[
            pl.BlockSpec((1, 3, N), lambda b: (b, 0, 0)),
            pl.BlockSpec((3, C1), lambda b: (0, 0)),
            pl.BlockSpec((1, C1), lambda b: (0, 0)),
            pl.BlockSpec((2 * C1, C2), lambda b: (0, 0)),
        ],
        out_specs=[
            pl.BlockSpec((1, N, C2), lambda b: (b, 0, 0)),
            pl.BlockSpec((1, N, C2), lambda b: (b, 0, 0)),
        ],
        out_shape=[
            jax.ShapeDtypeStruct((B, N, C2), jnp.float32),
            jax.ShapeDtypeStruct((B, N, C2), jnp.float32),
        ],
    )(f, W1, b1.reshape(1, C1), W2)


def _k2_body(xr_ref, x_ref, idx_ref):
    xr = xr_ref[0]
    xb = x_ref[0]
    g = lax.dot_general(xr, xb, (((1,), (0,)), ((), ())),
                        preferred_element_type=jnp.float32)
    xi2 = jnp.sum(xr * xr, axis=1, keepdims=True)
    xj2 = jnp.sum(xb * xb, axis=0, keepdims=True)
    cur = (-2.0 * g + xi2) + xj2
    iota_f = lax.broadcasted_iota(jnp.int32, (ROWS, N), 1).astype(jnp.float32)
    cols = []
    for _ in range(K):
        m = jnp.min(cur, axis=1, keepdims=True)
        cand = jnp.where(cur == m, iota_f, float(N))
        sel = jnp.min(cand, axis=1, keepdims=True)
        cols.append(sel.astype(jnp.int32))
        cur = jnp.where(iota_f == sel, jnp.inf, cur)
    idx_ref[0] = jnp.concatenate(cols, axis=1)


def _run_k2(x, xt):
    return pl.pallas_call(
        _k2_body,
        grid=(B, N // ROWS),
        in_specs=[
            pl.BlockSpec((1, ROWS, 3), lambda b, r: (b, r, 0)),
            pl.BlockSpec((1, 3, N), lambda b, r: (b, 0, 0)),
        ],
        out_specs=pl.BlockSpec((1, ROWS, K), lambda b, r: (b, r, 0)),
        out_shape=jax.ShapeDtypeStruct((B, N, K), jnp.int32),
    )(xt, x)


def _k3_body(x_ref, out_ref):
    x0 = x_ref[:, 0, :]
    x1 = x_ref[:, 1, :]
    x2 = x_ref[:, 2, :]
    x012 = jnp.concatenate([x0, x1, x2], axis=0)
    iota_f = lax.broadcasted_iota(jnp.int32, (B, N), 1).astype(jnp.float32)
    iota3_f = lax.broadcasted_iota(jnp.int32, (3 * B, N), 1
                                   ).astype(jnp.float32)
    iota_np = lax.broadcasted_iota(jnp.int32, (B, NPOINT), 1)

    def body(i, carry):
        dist, far, fars = carry
        far_mat = far.astype(jnp.int32) + 0 * iota_np
        fars = jnp.where(iota_np == i, far_mat, fars)
        far3 = jnp.concatenate([far, far, far], axis=0)
        oh3 = iota3_f == far3
        cen = jnp.sum(jnp.where(oh3, x012, 0.0), axis=1, keepdims=True)
        cx = cen[0:B]
        cy = cen[B:2 * B]
        cz = cen[2 * B:3 * B]
        dx = x0 - cx
        dy = x1 - cy
        dz = x2 - cz
        d = (dx * dx + dy * dy) + dz * dz
        dist = jnp.minimum(dist, d)
        m = jnp.max(dist, axis=1, keepdims=True)
        cand = jnp.where(dist == m, iota_f, float(N))
        far_new = jnp.min(cand, axis=1, keepdims=True)
        return dist, far_new, fars

    init = (1e10 + 0.0 * x0,
            0.0 * jnp.min(x0, axis=1, keepdims=True),
            (0.0 * x0[:, :NPOINT]).astype(jnp.int32))
    _, _, fars = lax.fori_loop(0, NPOINT, body, init)
    out_ref[...] = fars


def _run_k3(x):
    return pl.pallas_call(
        _k3_body,
        grid=(1,),
        in_specs=[pl.BlockSpec((B, 3, N), lambda i: (0, 0, 0))],
        out_specs=pl.BlockSpec((B, NPOINT), lambda i: (0, 0)),
        out_shape=jax.ShapeDtypeStruct((B, NPOINT), jnp.int32),
    )(x)


PB = 8
EB = PB * K


def _k4_sc_body(u_hbm, v_hbm, idx_hbm, mu_hbm, cross_hbm, sumu_hbm,
                sumusq_hbm, idx0, idx1, idxg0, idxg1, rows0, rows1, vb0, vb1,
                mu_buf, cross_v, sumu_v, sumusq_v, si0, si1, sd0, sd1):
    info = plsc.get_sparse_core_info()
    nc = info.num_cores
    w = lax.axis_index("s") * nc + lax.axis_index("c")
    ppw = (B * N) // (nc * info.num_subcores)
    base = (w // (N // ppw)) * N
    nblk = ppw // PB
    bufs = [(idx0, idxg0, rows0, vb0, si0, sd0),
            (idx1, idxg1, rows1, vb1, si1, sd1)]

    zero16f = jnp.zeros((16,), jnp.float32)
    for j in range(C2 // 16):
        cross_v[pl.ds(j * 16, 16)] = zero16f
        sumu_v[pl.ds(j * 16, 16)] = zero16f
        sumusq_v[pl.ds(j * 16, 16)] = zero16f

    def fire_idx(g, k):
        idx_v, _, _, _, si, _ = bufs[k]
        pltpu.async_copy(idx_hbm.at[pl.ds((w * ppw + g * PB) * K, EB)],
                         idx_v, si)

    def wait_idx(g, k):
        idx_v, _, _, _, si, _ = bufs[k]
        pltpu.make_async_copy(idx_hbm.at[pl.ds((w * ppw + g * PB) * K, EB)],
                              idx_v, si).wait()

    def fire_dat(g, k):
        idx_v, idxg_v, rows_v, v_buf, _, sd = bufs[k]
        for j in range(EB // 16):
            t = idx_v[pl.ds(j * 16, 16)]
            idxg_v[pl.ds(j * 16, 16)] = t + base
        pltpu.async_copy(u_hbm.at[idxg_v], rows_v, sd)
        pltpu.async_copy(v_hbm.at[pl.ds(w * ppw + g * PB, PB)], v_buf, sd)

    def wait_dat(g, k):
        _, idxg_v, rows_v, v_buf, _, sd = bufs[k]
        pltpu.make_async_copy(u_hbm.at[idxg_v], rows_v, sd).wait()
        pltpu.make_async_copy(v_hbm.at[pl.ds(w * ppw + g * PB, PB)], v_buf,
                              sd).wait()

    def compute(g, k):
        _, _, rows_v, v_buf, _, _ = bufs[k]

        def chunk_body(c, _):
            co = c * 16
            cacc = cross_v[pl.ds(co, 16)]
            uacc = sumu_v[pl.ds(co, 16)]
            qacc = sumusq_v[pl.ds(co, 16)]
            for p in range(PB):
                r0 = p * K
                m = rows_v[r0, pl.ds(co, 16)]
                s = m
                q = m * m
                for r in range(1, K):
                    t = rows_v[r0 + r, pl.ds(co, 16)]
                    m = jnp.maximum(m, t)
                    s = s + t
                    q = q + t * t
                mu_buf[p, pl.ds(co, 16)] = m
                cacc = cacc + s * v_buf[p, pl.ds(co, 16)]
                uacc = uacc + s
                qacc = qacc + q
            cross_v[pl.ds(co, 16)] = cacc
            sumu_v[pl.ds(co, 16)] = uacc
            sumusq_v[pl.ds(co, 16)] = qacc
            return 0

        lax.fori_loop(0, C2 // 16, chunk_body, 0)
        pltpu.sync_copy(mu_buf, mu_hbm.at[pl.ds(w * ppw + g * PB, PB)])

    fire_idx(0, 0)
    fire_idx(1, 1)
    wait_idx(0, 0)
    fire_dat(0, 0)

    def pipe_body(h, _):
        g0 = 2 * h
        g1 = g0 + 1
        wait_idx(g1, 1)
        fire_dat(g1, 1)

        @pl.when(g0 + 2 < nblk)
        def _():
            fire_idx(g0 + 2, 0)

        wait_dat(g0, 0)
        compute(g0, 0)

        @pl.when(g0 + 2 < nblk)
        def _():
            wait_idx(g0 + 2, 0)
            fire_dat(g0 + 2, 0)

        @pl.when(g1 + 2 < nblk)
        def _():
            fire_idx(g1 + 2, 1)

        wait_dat(g1, 1)
        compute(g1, 1)
        return 0

    lax.fori_loop(0, nblk // 2, pipe_body, 0)
    pltpu.sync_copy(cross_v, cross_hbm.at[w])
    pltpu.sync_copy(sumu_v, sumu_hbm.at[w])
    pltpu.sync_copy(sumusq_v, sumusq_hbm.at[w])


def _run_k4(u_flat, v_flat, idx_flat):
    mesh = plsc.VectorSubcoreMesh(core_axis_name="c", subcore_axis_name="s")
    kfn = functools.partial(
        pl.kernel,
        mesh=mesh,
        out_type=[
            jax.ShapeDtypeStruct((B * N, C2), jnp.float32),
            jax.ShapeDtypeStruct((32, C2), jnp.float32),
            jax.ShapeDtypeStruct((32, C2), jnp.float32),
            jax.ShapeDtypeStruct((32, C2), jnp.float32),
        ],
        scratch_types=[
            pltpu.VMEM((EB,), jnp.int32),
            pltpu.VMEM((EB,), jnp.int32),
            pltpu.VMEM((EB,), jnp.int32),
            pltpu.VMEM((EB,), jnp.int32),
            pltpu.VMEM((EB, C2), jnp.float32),
            pltpu.VMEM((EB, C2), jnp.float32),
            pltpu.VMEM((PB, C2), jnp.float32),
            pltpu.VMEM((PB, C2), jnp.float32),
            pltpu.VMEM((PB, C2), jnp.float32),
            pltpu.VMEM((C2,), jnp.float32),
            pltpu.VMEM((C2,), jnp.float32),
            pltpu.VMEM((C2,), jnp.float32),
            pltpu.SemaphoreType.DMA,
            pltpu.SemaphoreType.DMA,
            pltpu.SemaphoreType.DMA,
            pltpu.SemaphoreType.DMA,
        ],
    )(_k4_sc_body)
    return kfn(u_flat, v_flat, idx_flat)


def _k5_body(u_ref, v_ref, mu_ref, sumu_ref, sumusq_ref, cross_ref, fps_ref,
             x_ref, g_ref, bt_ref, coor_ref, newx_ref):
    v = v_ref[0]
    mu = mu_ref[0]
    cross = jnp.sum(cross_ref[0], axis=0, keepdims=True)
    hi = jax.lax.Precision.HIGHEST
    sum_u = jnp.sum(sumu_ref[0], axis=0, keepdims=True)
    sum_usq = jnp.sum(sumusq_ref[0], axis=0, keepdims=True)
    sum_v = jnp.sum(v, axis=0, keepdims=True)
    sum_vsq = jnp.sum(v * v, axis=0, keepdims=True)
    s1 = sum_u + float(K) * sum_v
    s2 = sum_usq + 2.0 * cross + float(K) * sum_vsq
    gsel = (lax.broadcasted_iota(jnp.int32, (C2, NGROUP), 0) // (C2 // NGROUP)
            == lax.broadcasted_iota(jnp.int32, (C2, NGROUP), 1)
            ).astype(jnp.float32)
    s1g = lax.dot_general(s1, gsel, (((1,), (0,)), ((), ())),
                          precision=hi, preferred_element_type=jnp.float32)
    s2g = lax.dot_general(s2, gsel, (((1,), (0,)), ((), ())),
                          precision=hi, preferred_element_type=jnp.float32)
    cnt_total = float((C2 // NGROUP) * N * K)
    mean_g = s1g / cnt_total
    var_g = s2g / cnt_total - mean_g * mean_g
    rstd_g = lax.rsqrt(var_g + 1e-5)
    gselt = (lax.broadcasted_iota(jnp.int32, (NGROUP, C2), 0) ==
             lax.broadcasted_iota(jnp.int32, (NGROUP, C2), 1) // (C2 // NGROUP)
             ).astype(jnp.float32)
    rstd = lax.dot_general(rstd_g, gselt, (((1,), (0,)), ((), ())),
                           precision=hi, preferred_element_type=jnp.float32)
    mean = lax.dot_general(mean_g, gselt, (((1,), (0,)), ((), ())),
                           precision=hi, preferred_element_type=jnp.float32)
    scale = g_ref[...] * rstd
    shift = bt_ref[...] - mean * scale
    act = (mu + v) * scale + shift
    act = jnp.where(act >= 0.0, act, 0.2 * act)
    frow = fps_ref[0]
    pt = (lax.broadcasted_iota(jnp.int32, (N, NPOINT), 0) == frow
          ).astype(jnp.float32)
    newx_ref[0] = lax.dot_general(act, pt, (((0,), (0,)), ((), ())),
                                  precision=hi,
                                  preferred_element_type=jnp.float32)
    xb = x_ref[0]
    coor_ref[0] = lax.dot_general(xb, pt, (((1,), (0,)), ((), ())),
                                  precision=hi,
                                  preferred_element_type=jnp.float32)


def _run_k5(u, v, mu, sumu_part, sumusq_part, cross_part, fps_sm, x, gamma,
            beta):
    return pl.pallas_call(
        _k5_body,
        grid=(B,),
        in_specs=[
            pl.BlockSpec((1, N, C2), lambda b: (b, 0, 0)),
            pl.BlockSpec((1, N, C2), lambda b: (b, 0, 0)),
            pl.BlockSpec((1, N, C2), lambda b: (b, 0, 0)),
            pl.BlockSpec((1, 4, C2), lambda b: (b, 0, 0)),
            pl.BlockSpec((1, 4, C2), lambda b: (b, 0, 0)),
            pl.BlockSpec((1, 4, C2), lambda b: (b, 0, 0)),
            pl.BlockSpec((1, 1, NPOINT), lambda b: (b, 0, 0)),
            pl.BlockSpec((1, 3, N), lambda b: (b, 0, 0)),
            pl.BlockSpec((1, C2), lambda b: (0, 0)),
            pl.BlockSpec((1, C2), lambda b: (0, 0)),
        ],
        out_specs=[
            pl.BlockSpec((1, 3, NPOINT), lambda b: (b, 0, 0)),
            pl.BlockSpec((1, C2, NPOINT), lambda b: (b, 0, 0)),
        ],
        out_shape=[
            jax.ShapeDtypeStruct((B, 3, NPOINT), jnp.float32),
            jax.ShapeDtypeStruct((B, C2, NPOINT), jnp.float32),
        ],
    )(u, v, mu, sumu_part.reshape(B, 4, C2), sumusq_part.reshape(B, 4, C2),
      cross_part.reshape(B, 4, C2), fps_sm.reshape(B, 1, NPOINT), x,
      gamma.reshape(1, C2), beta.reshape(1, C2))


def kernel(x, f, W1, b1, W2, gamma, beta):
    xt = jnp.swapaxes(x, 1, 2)
    u, v = _run_k1(f, W1, b1, W2)
    idx = _run_k2(x, xt)
    mu_flat, cross_part, sumu_part, sumusq_part = _run_k4(
        u.reshape(B * N, C2), v.reshape(B * N, C2),
        idx.reshape(B * N * K))
    fps_sm = _run_k3(x)
    mu = mu_flat.reshape(B, N, C2)
    new_coor, new_x = _run_k5(u, v, mu, sumu_part, sumusq_part, cross_part,
                              fps_sm, x, gamma, beta)
    return (new_coor, new_x, fps_sm)
